# Initial kernel scaffold; baseline (speedup 1.0000x reference)
import jax
import jax.numpy as jnp
from jax.experimental import pallas as pl


def kernel(edge_index, user_emb, movie_emb, Wl1, bl1, Wr1, Wl2, bl2, Wr2):
    raise NotImplementedError("write your pallas kernel here")


# speedup vs baseline: 4.3676x; 4.3676x over previous
import functools

import jax
import jax.numpy as jnp
from jax import lax
from jax.experimental import pallas as pl
from jax.experimental.pallas import tpu as pltpu
from jax.experimental.pallas import tpu_sc as plsc

N_USERS = 25000
N_NODES = 50000
E = 800000
IN_CH = 64
HID = 128
OUT = 64

NT = 16
ROWS = 50048
RPT = ROWS // NT
CHUNK = 128
GC = 392
EPT = CHUNK * GC
E_PAD = EPT * NT
HALF = 32

_MESH = plsc.VectorSubcoreMesh(core_axis_name="c", subcore_axis_name="s")

NW = 2 * NT
GC2 = E_PAD // (CHUNK * NW)


def _sc_pass_body(table, s_st, d2, z32, out2, acc, si0, di0, rows0, g0):
    c = lax.axis_index("c")
    t = lax.axis_index("s")
    tslice = pl.ds(t * RPT, RPT)

    pltpu.sync_copy(z32, acc.at[tslice])
    plsc.subcore_barrier()

    @pl.loop(0, GC)
    def _(chunk):
        rb = t * GC + chunk
        pltpu.sync_copy(s_st.at[c, rb], si0)
        pltpu.sync_copy(d2.at[rb], di0)
        pltpu.async_copy(table.at[si0], rows0, g0).wait()
        pltpu.sync_copy(rows0, acc.at[di0], add=True)

    plsc.subcore_barrier()
    pltpu.sync_copy(acc.at[tslice], out2.at[c, tslice])


_sc_pass = pl.kernel(
    _sc_pass_body,
    out_type=[jax.ShapeDtypeStruct((2, ROWS, HALF), jnp.float32)],
    mesh=_MESH,
    scratch_types=[
        pltpu.VMEM_SHARED((ROWS, HALF), jnp.float32),
        pltpu.VMEM((CHUNK,), jnp.int32),
        pltpu.VMEM((CHUNK,), jnp.int32),
        pltpu.VMEM((CHUNK, HALF), jnp.float32),
        pltpu.SemaphoreType.DMA,
    ],
    compiler_params=pltpu.CompilerParams(use_tc_tiling_on_sc=False),
    name="sage_seg_sum",
)


def _sc_count_body(d2, z16, o16, cntout, cnt, di0, ones_v):
    c = lax.axis_index("c")
    t = lax.axis_index("s")
    tslice = pl.ds(t * RPT, RPT)

    pltpu.sync_copy(z16, cnt.at[tslice])
    pltpu.sync_copy(o16, ones_v)
    plsc.subcore_barrier()

    @pl.loop(0, GC2)
    def _(chunk):
        rb = (c * NT + t) * GC2 + chunk
        pltpu.sync_copy(d2.at[rb], di0)
        pltpu.sync_copy(ones_v, cnt.at[di0], add=True)

    plsc.subcore_barrier()
    pltpu.sync_copy(cnt.at[tslice], cntout.at[c, tslice])


_sc_count = pl.kernel(
    _sc_count_body,
    out_type=[jax.ShapeDtypeStruct((2, ROWS, 16), jnp.float32)],
    mesh=_MESH,
    scratch_types=[
        pltpu.VMEM_SHARED((ROWS, 16), jnp.float32),
        pltpu.VMEM((CHUNK,), jnp.int32),
        pltpu.VMEM((CHUNK, 16), jnp.float32),
    ],
    compiler_params=pltpu.CompilerParams(use_tc_tiling_on_sc=False),
    name="sage_seg_count",
)


_BT = 3128
_TCGRID = ROWS // _BT


def _tc1_body(agg_ref, cnt_ref, x_ref, wl1_ref, bl1_ref, wr1_ref, wl2_ref,
              h1_ref, y2_ref):
    inv = 1.0 / jnp.maximum(cnt_ref[0, :, 0:1] + cnt_ref[1, :, 0:1], 1.0)
    mean = jnp.concatenate([agg_ref[0], agg_ref[1]], axis=1) * inv
    pre = (jnp.dot(mean, wl1_ref[...], preferred_element_type=jnp.float32)
           + bl1_ref[...]
           + jnp.dot(x_ref[...], wr1_ref[...], preferred_element_type=jnp.float32))
    h1 = jnp.maximum(pre, 0.0)
    h1_ref[...] = h1
    y = jnp.dot(h1, wl2_ref[...], preferred_element_type=jnp.float32)
    y2_ref[0] = y[:, :HALF]
    y2_ref[1] = y[:, HALF:]


_tc1 = pl.pallas_call(
    _tc1_body,
    grid=(_TCGRID,),
    in_specs=[
        pl.BlockSpec((2, _BT, HALF), lambda i: (0, i, 0)),
        pl.BlockSpec((2, _BT, 16), lambda i: (0, i, 0)),
        pl.BlockSpec((_BT, IN_CH), lambda i: (i, 0)),
        pl.BlockSpec((IN_CH, HID), lambda i: (0, 0)),
        pl.BlockSpec((1, HID), lambda i: (0, 0)),
        pl.BlockSpec((IN_CH, HID), lambda i: (0, 0)),
        pl.BlockSpec((HID, OUT), lambda i: (0, 0)),
    ],
    out_specs=[
        pl.BlockSpec((_BT, HID), lambda i: (i, 0)),
        pl.BlockSpec((2, _BT, HALF), lambda i: (0, i, 0)),
    ],
    out_shape=[
        jax.ShapeDtypeStruct((ROWS, HID), jnp.float32),
        jax.ShapeDtypeStruct((2, ROWS, HALF), jnp.float32),
    ],
)


def _tc2_body(agg_ref, cnt_ref, h1_ref, wr2_ref, bl2_ref, out_ref):
    inv = 1.0 / jnp.maximum(cnt_ref[0, :, 0:1] + cnt_ref[1, :, 0:1], 1.0)
    mean = jnp.concatenate([agg_ref[0], agg_ref[1]], axis=1) * inv
    out_ref[...] = (mean + bl2_ref[...]
                    + jnp.dot(h1_ref[...], wr2_ref[...],
                              preferred_element_type=jnp.float32))


_tc2 = pl.pallas_call(
    _tc2_body,
    grid=(_TCGRID,),
    in_specs=[
        pl.BlockSpec((2, _BT, HALF), lambda i: (0, i, 0)),
        pl.BlockSpec((2, _BT, 16), lambda i: (0, i, 0)),
        pl.BlockSpec((_BT, HID), lambda i: (i, 0)),
        pl.BlockSpec((HID, OUT), lambda i: (0, 0)),
        pl.BlockSpec((1, OUT), lambda i: (0, 0)),
    ],
    out_specs=pl.BlockSpec((_BT, OUT), lambda i: (i, 0)),
    out_shape=jax.ShapeDtypeStruct((ROWS, OUT), jnp.float32),
)


@jax.jit
def kernel(edge_index, user_emb, movie_emb, Wl1, bl1, Wr1, Wl2, bl2, Wr2):
    f32 = jnp.float32
    src = edge_index[0]
    dst = edge_index[1]

    npad = E_PAD - E
    ar = jnp.arange(npad, dtype=jnp.int32)
    src_p = jnp.concatenate([src, (ar * 997) % N_NODES])
    dst_p = jnp.concatenate([dst, N_NODES + (ar % (ROWS - N_NODES))])
    src2a = src_p.reshape(-1, CHUNK)
    dst2 = dst_p.reshape(-1, CHUNK)

    x = jnp.concatenate(
        [user_emb, movie_emb, jnp.zeros((ROWS - N_NODES, IN_CH), f32)], axis=0)
    x2 = jnp.stack([x[:, :HALF], x[:, HALF:]]).reshape(2 * ROWS, HALF)

    z32 = jnp.zeros((RPT, HALF), f32)
    z16 = jnp.zeros((RPT, 16), f32)
    o16 = jnp.ones((CHUNK, 16), f32)

    s_st = jnp.stack([src2a, src2a + ROWS])
    (cnt8,) = _sc_count(dst2, z16, o16)
    (agg1,) = _sc_pass(x2, s_st, dst2, z32)
    h1, y2 = _tc1(agg1, cnt8, x, Wl1.T, bl1.reshape(1, -1), Wr1.T, Wl2.T)
    (agg2,) = _sc_pass(y2.reshape(2 * ROWS, HALF), s_st, dst2, z32)
    out = _tc2(agg2, cnt8, h1, Wr2.T, bl2.reshape(1, -1))
    return out[:N_USERS], out[N_USERS:N_NODES]


# speedup vs baseline: 5.0798x; 1.1631x over previous
import functools

import jax
import jax.numpy as jnp
from jax import lax
from jax.experimental import pallas as pl
from jax.experimental.pallas import tpu as pltpu
from jax.experimental.pallas import tpu_sc as plsc

N_USERS = 25000
N_NODES = 50000
E = 800000
IN_CH = 64
HID = 128
OUT = 64

NT = 16
ROWS = 50048
RPT = ROWS // NT
CHUNK = 128
GC = 392
EPT = CHUNK * GC
E_PAD = EPT * NT
HALF = 32

_MESH = plsc.VectorSubcoreMesh(core_axis_name="c", subcore_axis_name="s")

NW = 2 * NT
GC2 = E_PAD // (CHUNK * NW)


_NB = 2


def _sc_pass_body(table, s_st, d2, z32, out2, acc, *scr):
    sis = scr[0:_NB]
    dis = scr[_NB:2 * _NB]
    rows = scr[2 * _NB:3 * _NB]
    gi, gr = scr[3 * _NB], scr[3 * _NB + 1]

    c = lax.axis_index("c")
    t = lax.axis_index("s")
    tslice = pl.ds(t * RPT, RPT)

    pltpu.sync_copy(z32, acc.at[tslice])
    plsc.subcore_barrier()

    @pl.loop(0, GC // _NB)
    def _(i):
        rb = t * GC + _NB * i
        for b in range(_NB):
            pltpu.sync_copy(s_st.at[c, rb + b], sis[b])
            pltpu.sync_copy(d2.at[rb + b], dis[b])
        gdescs = []
        for b in range(_NB):
            gdescs.append(pltpu.async_copy(table.at[sis[b]], rows[b], gr))
        for b in range(_NB):
            gdescs[b].wait()
            pltpu.sync_copy(rows[b], acc.at[dis[b]], add=True)

    plsc.subcore_barrier()
    pltpu.sync_copy(acc.at[tslice], out2.at[c, tslice])


_sc_pass = pl.kernel(
    _sc_pass_body,
    out_type=[jax.ShapeDtypeStruct((2, ROWS, HALF), jnp.float32)],
    mesh=_MESH,
    scratch_types=(
        [pltpu.VMEM_SHARED((ROWS, HALF), jnp.float32)]
        + [pltpu.VMEM((CHUNK,), jnp.int32) for _ in range(2 * _NB)]
        + [pltpu.VMEM((CHUNK, HALF), jnp.float32) for _ in range(_NB)]
        + [pltpu.SemaphoreType.DMA, pltpu.SemaphoreType.DMA]
    ),
    compiler_params=pltpu.CompilerParams(use_tc_tiling_on_sc=False),
    name="sage_seg_sum",
)


def _sc_count_body(d2, z16, o16, cntout, cnt, di0, ones_v):
    c = lax.axis_index("c")
    t = lax.axis_index("s")
    tslice = pl.ds(t * RPT, RPT)

    pltpu.sync_copy(z16, cnt.at[tslice])
    pltpu.sync_copy(o16, ones_v)
    plsc.subcore_barrier()

    @pl.loop(0, GC2)
    def _(chunk):
        rb = (c * NT + t) * GC2 + chunk
        pltpu.sync_copy(d2.at[rb], di0)
        pltpu.sync_copy(ones_v, cnt.at[di0], add=True)

    plsc.subcore_barrier()
    pltpu.sync_copy(cnt.at[tslice], cntout.at[c, tslice])


_sc_count = pl.kernel(
    _sc_count_body,
    out_type=[jax.ShapeDtypeStruct((2, ROWS, 16), jnp.float32)],
    mesh=_MESH,
    scratch_types=[
        pltpu.VMEM_SHARED((ROWS, 16), jnp.float32),
        pltpu.VMEM((CHUNK,), jnp.int32),
        pltpu.VMEM((CHUNK, 16), jnp.float32),
    ],
    compiler_params=pltpu.CompilerParams(use_tc_tiling_on_sc=False),
    name="sage_seg_count",
)


_BT = 3128
_TCGRID = ROWS // _BT


def _tc1_body(agg_ref, cnt_ref, x_ref, wl1_ref, bl1_ref, wr1_ref, wl2_ref,
              h1_ref, y2_ref):
    inv = 1.0 / jnp.maximum(cnt_ref[0, :, 0:1] + cnt_ref[1, :, 0:1], 1.0)
    mean = jnp.concatenate([agg_ref[0], agg_ref[1]], axis=1) * inv
    pre = (jnp.dot(mean, wl1_ref[...], preferred_element_type=jnp.float32)
           + bl1_ref[...]
           + jnp.dot(x_ref[...], wr1_ref[...], preferred_element_type=jnp.float32))
    h1 = jnp.maximum(pre, 0.0)
    h1_ref[...] = h1
    y = jnp.dot(h1, wl2_ref[...], preferred_element_type=jnp.float32)
    y2_ref[0] = y[:, :HALF]
    y2_ref[1] = y[:, HALF:]


_tc1 = pl.pallas_call(
    _tc1_body,
    grid=(_TCGRID,),
    in_specs=[
        pl.BlockSpec((2, _BT, HALF), lambda i: (0, i, 0)),
        pl.BlockSpec((2, _BT, 16), lambda i: (0, i, 0)),
        pl.BlockSpec((_BT, IN_CH), lambda i: (i, 0)),
        pl.BlockSpec((IN_CH, HID), lambda i: (0, 0)),
        pl.BlockSpec((1, HID), lambda i: (0, 0)),
        pl.BlockSpec((IN_CH, HID), lambda i: (0, 0)),
        pl.BlockSpec((HID, OUT), lambda i: (0, 0)),
    ],
    out_specs=[
        pl.BlockSpec((_BT, HID), lambda i: (i, 0)),
        pl.BlockSpec((2, _BT, HALF), lambda i: (0, i, 0)),
    ],
    out_shape=[
        jax.ShapeDtypeStruct((ROWS, HID), jnp.float32),
        jax.ShapeDtypeStruct((2, ROWS, HALF), jnp.float32),
    ],
)


def _tc2_body(agg_ref, cnt_ref, h1_ref, wr2_ref, bl2_ref, out_ref):
    inv = 1.0 / jnp.maximum(cnt_ref[0, :, 0:1] + cnt_ref[1, :, 0:1], 1.0)
    mean = jnp.concatenate([agg_ref[0], agg_ref[1]], axis=1) * inv
    out_ref[...] = (mean + bl2_ref[...]
                    + jnp.dot(h1_ref[...], wr2_ref[...],
                              preferred_element_type=jnp.float32))


_tc2 = pl.pallas_call(
    _tc2_body,
    grid=(_TCGRID,),
    in_specs=[
        pl.BlockSpec((2, _BT, HALF), lambda i: (0, i, 0)),
        pl.BlockSpec((2, _BT, 16), lambda i: (0, i, 0)),
        pl.BlockSpec((_BT, HID), lambda i: (i, 0)),
        pl.BlockSpec((HID, OUT), lambda i: (0, 0)),
        pl.BlockSpec((1, OUT), lambda i: (0, 0)),
    ],
    out_specs=pl.BlockSpec((_BT, OUT), lambda i: (i, 0)),
    out_shape=jax.ShapeDtypeStruct((ROWS, OUT), jnp.float32),
)


@jax.jit
def kernel(edge_index, user_emb, movie_emb, Wl1, bl1, Wr1, Wl2, bl2, Wr2):
    f32 = jnp.float32
    src = edge_index[0]
    dst = edge_index[1]

    npad = E_PAD - E
    ar = jnp.arange(npad, dtype=jnp.int32)
    src_p = jnp.concatenate([src, (ar * 997) % N_NODES])
    dst_p = jnp.concatenate([dst, N_NODES + (ar % (ROWS - N_NODES))])
    src2a = src_p.reshape(-1, CHUNK)
    dst2 = dst_p.reshape(-1, CHUNK)

    x = jnp.concatenate(
        [user_emb, movie_emb, jnp.zeros((ROWS - N_NODES, IN_CH), f32)], axis=0)
    x2 = jnp.stack([x[:, :HALF], x[:, HALF:]]).reshape(2 * ROWS, HALF)

    z32 = jnp.zeros((RPT, HALF), f32)
    z16 = jnp.zeros((RPT, 16), f32)
    o16 = jnp.ones((CHUNK, 16), f32)

    s_st = jnp.stack([src2a, src2a + ROWS])
    (cnt8,) = _sc_count(dst2, z16, o16)
    (agg1,) = _sc_pass(x2, s_st, dst2, z32)
    h1, y2 = _tc1(agg1, cnt8, x, Wl1.T, bl1.reshape(1, -1), Wr1.T, Wl2.T)
    (agg2,) = _sc_pass(y2.reshape(2 * ROWS, HALF), s_st, dst2, z32)
    out = _tc2(agg2, cnt8, h1, Wr2.T, bl2.reshape(1, -1))
    return out[:N_USERS], out[N_USERS:N_NODES]


# speedup vs baseline: 7.1866x; 1.4148x over previous
import functools

import jax
import jax.numpy as jnp
from jax import lax
from jax.experimental import pallas as pl
from jax.experimental.pallas import tpu as pltpu
from jax.experimental.pallas import tpu_sc as plsc

N_USERS = 25000
N_NODES = 50000
E = 800000
IN_CH = 64
HID = 128
OUT = 64

NT = 16
ROWS = 50048
RPT = ROWS // NT
CHUNK = 256
GC = 196
EPT = CHUNK * GC
E_PAD = EPT * NT
HALF = 32

_MESH = plsc.VectorSubcoreMesh(core_axis_name="c", subcore_axis_name="s")

NW = 2 * NT
GC2 = E_PAD // (CHUNK * NW)


_NB = 2


def _sc_pass_body(table, s_st, d2, z32, out2, acc, *scr):
    sis = scr[0:_NB]
    dis = scr[_NB:2 * _NB]
    rows = scr[2 * _NB:3 * _NB]
    gi, gr = scr[3 * _NB], scr[3 * _NB + 1]

    c = lax.axis_index("c")
    t = lax.axis_index("s")
    tslice = pl.ds(t * RPT, RPT)

    pltpu.sync_copy(z32, acc.at[tslice])
    plsc.subcore_barrier()

    @pl.loop(0, GC // _NB)
    def _(i):
        rb = t * GC + _NB * i
        for b in range(_NB):
            pltpu.sync_copy(s_st.at[c, rb + b], sis[b])
            pltpu.sync_copy(d2.at[rb + b], dis[b])
        gdescs = []
        for b in range(_NB):
            gdescs.append(pltpu.async_copy(table.at[sis[b]], rows[b], gr))
        for b in range(_NB):
            gdescs[b].wait()
            pltpu.sync_copy(rows[b], acc.at[dis[b]], add=True)

    plsc.subcore_barrier()
    pltpu.sync_copy(acc.at[tslice], out2.at[c, tslice])


_sc_pass = pl.kernel(
    _sc_pass_body,
    out_type=[jax.ShapeDtypeStruct((2, ROWS, HALF), jnp.float32)],
    mesh=_MESH,
    scratch_types=(
        [pltpu.VMEM_SHARED((ROWS, HALF), jnp.float32)]
        + [pltpu.VMEM((CHUNK,), jnp.int32) for _ in range(2 * _NB)]
        + [pltpu.VMEM((CHUNK, HALF), jnp.float32) for _ in range(_NB)]
        + [pltpu.SemaphoreType.DMA, pltpu.SemaphoreType.DMA]
    ),
    compiler_params=pltpu.CompilerParams(use_tc_tiling_on_sc=False),
    name="sage_seg_sum",
)


def _sc_count_body(d2, z16, o16, cntout, cnt, di0, ones_v):
    c = lax.axis_index("c")
    t = lax.axis_index("s")
    tslice = pl.ds(t * RPT, RPT)

    pltpu.sync_copy(z16, cnt.at[tslice])
    pltpu.sync_copy(o16, ones_v)
    plsc.subcore_barrier()

    @pl.loop(0, GC2)
    def _(chunk):
        rb = (c * NT + t) * GC2 + chunk
        pltpu.sync_copy(d2.at[rb], di0)
        pltpu.sync_copy(ones_v, cnt.at[di0], add=True)

    plsc.subcore_barrier()
    pltpu.sync_copy(cnt.at[tslice], cntout.at[c, tslice])


_sc_count = pl.kernel(
    _sc_count_body,
    out_type=[jax.ShapeDtypeStruct((2, ROWS, 16), jnp.float32)],
    mesh=_MESH,
    scratch_types=[
        pltpu.VMEM_SHARED((ROWS, 16), jnp.float32),
        pltpu.VMEM((CHUNK,), jnp.int32),
        pltpu.VMEM((CHUNK, 16), jnp.float32),
    ],
    compiler_params=pltpu.CompilerParams(use_tc_tiling_on_sc=False),
    name="sage_seg_count",
)


_BT = 3128
_TCGRID = ROWS // _BT


def _tc1_body(agg_ref, cnt_ref, x_ref, wl1_ref, bl1_ref, wr1_ref, wl2_ref,
              h1_ref, y2_ref):
    inv = 1.0 / jnp.maximum(cnt_ref[0, :, 0:1] + cnt_ref[1, :, 0:1], 1.0)
    mean = jnp.concatenate([agg_ref[0], agg_ref[1]], axis=1) * inv
    pre = (jnp.dot(mean, wl1_ref[...], preferred_element_type=jnp.float32)
           + bl1_ref[...]
           + jnp.dot(x_ref[...], wr1_ref[...], preferred_element_type=jnp.float32))
    h1 = jnp.maximum(pre, 0.0)
    h1_ref[...] = h1
    y = jnp.dot(h1, wl2_ref[...], preferred_element_type=jnp.float32)
    y2_ref[0] = y[:, :HALF]
    y2_ref[1] = y[:, HALF:]


_tc1 = pl.pallas_call(
    _tc1_body,
    grid=(_TCGRID,),
    in_specs=[
        pl.BlockSpec((2, _BT, HALF), lambda i: (0, i, 0)),
        pl.BlockSpec((2, _BT, 16), lambda i: (0, i, 0)),
        pl.BlockSpec((_BT, IN_CH), lambda i: (i, 0)),
        pl.BlockSpec((IN_CH, HID), lambda i: (0, 0)),
        pl.BlockSpec((1, HID), lambda i: (0, 0)),
        pl.BlockSpec((IN_CH, HID), lambda i: (0, 0)),
        pl.BlockSpec((HID, OUT), lambda i: (0, 0)),
    ],
    out_specs=[
        pl.BlockSpec((_BT, HID), lambda i: (i, 0)),
        pl.BlockSpec((2, _BT, HALF), lambda i: (0, i, 0)),
    ],
    out_shape=[
        jax.ShapeDtypeStruct((ROWS, HID), jnp.float32),
        jax.ShapeDtypeStruct((2, ROWS, HALF), jnp.float32),
    ],
)


def _tc2_body(agg_ref, cnt_ref, h1_ref, wr2_ref, bl2_ref, out_ref):
    inv = 1.0 / jnp.maximum(cnt_ref[0, :, 0:1] + cnt_ref[1, :, 0:1], 1.0)
    mean = jnp.concatenate([agg_ref[0], agg_ref[1]], axis=1) * inv
    out_ref[...] = (mean + bl2_ref[...]
                    + jnp.dot(h1_ref[...], wr2_ref[...],
                              preferred_element_type=jnp.float32))


_tc2 = pl.pallas_call(
    _tc2_body,
    grid=(_TCGRID,),
    in_specs=[
        pl.BlockSpec((2, _BT, HALF), lambda i: (0, i, 0)),
        pl.BlockSpec((2, _BT, 16), lambda i: (0, i, 0)),
        pl.BlockSpec((_BT, HID), lambda i: (i, 0)),
        pl.BlockSpec((HID, OUT), lambda i: (0, 0)),
        pl.BlockSpec((1, OUT), lambda i: (0, 0)),
    ],
    out_specs=pl.BlockSpec((_BT, OUT), lambda i: (i, 0)),
    out_shape=jax.ShapeDtypeStruct((ROWS, OUT), jnp.float32),
)


@jax.jit
def kernel(edge_index, user_emb, movie_emb, Wl1, bl1, Wr1, Wl2, bl2, Wr2):
    f32 = jnp.float32
    src = edge_index[0]
    dst = edge_index[1]

    npad = E_PAD - E
    ar = jnp.arange(npad, dtype=jnp.int32)
    src_p = jnp.concatenate([src, (ar * 997) % N_NODES])
    dst_p = jnp.concatenate([dst, N_NODES + (ar % (ROWS - N_NODES))])
    src2a = src_p.reshape(-1, CHUNK)
    dst2 = dst_p.reshape(-1, CHUNK)

    x = jnp.concatenate(
        [user_emb, movie_emb, jnp.zeros((ROWS - N_NODES, IN_CH), f32)], axis=0)
    x2 = jnp.stack([x[:, :HALF], x[:, HALF:]]).reshape(2 * ROWS, HALF)

    z32 = jnp.zeros((RPT, HALF), f32)
    z16 = jnp.zeros((RPT, 16), f32)
    o16 = jnp.ones((CHUNK, 16), f32)

    s_st = jnp.stack([src2a, src2a + ROWS])
    (cnt8,) = _sc_count(dst2, z16, o16)
    (agg1,) = _sc_pass(x2, s_st, dst2, z32)
    h1, y2 = _tc1(agg1, cnt8, x, Wl1.T, bl1.reshape(1, -1), Wr1.T, Wl2.T)
    (agg2,) = _sc_pass(y2.reshape(2 * ROWS, HALF), s_st, dst2, z32)
    out = _tc2(agg2, cnt8, h1, Wr2.T, bl2.reshape(1, -1))
    return out[:N_USERS], out[N_USERS:N_NODES]


# speedup vs baseline: 10.0195x; 1.3942x over previous
import functools

import jax
import jax.numpy as jnp
from jax import lax
from jax.experimental import pallas as pl
from jax.experimental.pallas import tpu as pltpu
from jax.experimental.pallas import tpu_sc as plsc

N_USERS = 25000
N_NODES = 50000
E = 800000
IN_CH = 64
HID = 128
OUT = 64

NT = 16
ROWS = 50048
RPT = ROWS // NT
CHUNK = 256
GC = 196
EPT = CHUNK * GC
E_PAD = EPT * NT
HALF = 32

_MESH = plsc.VectorSubcoreMesh(core_axis_name="c", subcore_axis_name="s")

NW = 2 * NT
GC2 = E_PAD // (CHUNK * NW)


_NB = 2


def _sc_pass_body(table, s_st, d2, z32, out2, acc, *scr):
    sis = scr[0:_NB]
    dis = scr[_NB:2 * _NB]
    rows = scr[2 * _NB:3 * _NB]
    gis = scr[3 * _NB:4 * _NB]
    gr = scr[4 * _NB]

    c = lax.axis_index("c")
    t = lax.axis_index("s")
    tslice = pl.ds(t * RPT, RPT)

    pltpu.sync_copy(z32, acc.at[tslice])
    plsc.subcore_barrier()

    NR = GC // _NB
    base = t * GC
    for b in range(_NB):
        pltpu.async_copy(s_st.at[c, base + b], sis[b], gis[b])
        pltpu.async_copy(d2.at[base + b], dis[b], gis[b])

    @pl.loop(0, NR)
    def _(i):
        rb = base + _NB * i
        gdescs = []
        for b in range(_NB):
            pltpu.make_async_copy(s_st.at[c, rb + b], sis[b], gis[b]).wait()
            pltpu.make_async_copy(d2.at[rb + b], dis[b], gis[b]).wait()
            gdescs.append(pltpu.async_copy(table.at[sis[b]], rows[b], gr))
        for b in range(_NB):
            gdescs[b].wait()
            pltpu.sync_copy(rows[b], acc.at[dis[b]], add=True)

            @pl.when(i < NR - 1)
            def _():
                rb2 = rb + _NB
                pltpu.async_copy(s_st.at[c, rb2 + b], sis[b], gis[b])
                pltpu.async_copy(d2.at[rb2 + b], dis[b], gis[b])

    plsc.subcore_barrier()
    pltpu.sync_copy(acc.at[tslice], out2.at[c, tslice])


_sc_pass = pl.kernel(
    _sc_pass_body,
    out_type=[jax.ShapeDtypeStruct((2, ROWS, HALF), jnp.float32)],
    mesh=_MESH,
    scratch_types=(
        [pltpu.VMEM_SHARED((ROWS, HALF), jnp.float32)]
        + [pltpu.VMEM((CHUNK,), jnp.int32) for _ in range(2 * _NB)]
        + [pltpu.VMEM((CHUNK, HALF), jnp.float32) for _ in range(_NB)]
        + [pltpu.SemaphoreType.DMA for _ in range(_NB + 1)]
    ),
    compiler_params=pltpu.CompilerParams(use_tc_tiling_on_sc=False),
    name="sage_seg_sum",
)


def _sc_count_body(d2, z16, o16, cntout, cnt, di0, ones_v):
    c = lax.axis_index("c")
    t = lax.axis_index("s")
    tslice = pl.ds(t * RPT, RPT)

    pltpu.sync_copy(z16, cnt.at[tslice])
    pltpu.sync_copy(o16, ones_v)
    plsc.subcore_barrier()

    @pl.loop(0, GC2)
    def _(chunk):
        rb = (c * NT + t) * GC2 + chunk
        pltpu.sync_copy(d2.at[rb], di0)
        pltpu.sync_copy(ones_v, cnt.at[di0], add=True)

    plsc.subcore_barrier()
    pltpu.sync_copy(cnt.at[tslice], cntout.at[c, tslice])


_sc_count = pl.kernel(
    _sc_count_body,
    out_type=[jax.ShapeDtypeStruct((2, ROWS, 16), jnp.float32)],
    mesh=_MESH,
    scratch_types=[
        pltpu.VMEM_SHARED((ROWS, 16), jnp.float32),
        pltpu.VMEM((CHUNK,), jnp.int32),
        pltpu.VMEM((CHUNK, 16), jnp.float32),
    ],
    compiler_params=pltpu.CompilerParams(use_tc_tiling_on_sc=False),
    name="sage_seg_count",
)


_BT = 3128
_TCGRID = ROWS // _BT


def _tc1_body(agg_ref, cnt_ref, x_ref, wl1_ref, bl1_ref, wr1_ref, wl2_ref,
              h1_ref, y2_ref):
    inv = 1.0 / jnp.maximum(cnt_ref[0, :, 0:1] + cnt_ref[1, :, 0:1], 1.0)
    mean = jnp.concatenate([agg_ref[0], agg_ref[1]], axis=1) * inv
    pre = (jnp.dot(mean, wl1_ref[...], preferred_element_type=jnp.float32)
           + bl1_ref[...]
           + jnp.dot(x_ref[...], wr1_ref[...], preferred_element_type=jnp.float32))
    h1 = jnp.maximum(pre, 0.0)
    h1_ref[...] = h1
    y = jnp.dot(h1, wl2_ref[...], preferred_element_type=jnp.float32)
    y2_ref[0] = y[:, :HALF]
    y2_ref[1] = y[:, HALF:]


_tc1 = pl.pallas_call(
    _tc1_body,
    grid=(_TCGRID,),
    in_specs=[
        pl.BlockSpec((2, _BT, HALF), lambda i: (0, i, 0)),
        pl.BlockSpec((2, _BT, 16), lambda i: (0, i, 0)),
        pl.BlockSpec((_BT, IN_CH), lambda i: (i, 0)),
        pl.BlockSpec((IN_CH, HID), lambda i: (0, 0)),
        pl.BlockSpec((1, HID), lambda i: (0, 0)),
        pl.BlockSpec((IN_CH, HID), lambda i: (0, 0)),
        pl.BlockSpec((HID, OUT), lambda i: (0, 0)),
    ],
    out_specs=[
        pl.BlockSpec((_BT, HID), lambda i: (i, 0)),
        pl.BlockSpec((2, _BT, HALF), lambda i: (0, i, 0)),
    ],
    out_shape=[
        jax.ShapeDtypeStruct((ROWS, HID), jnp.float32),
        jax.ShapeDtypeStruct((2, ROWS, HALF), jnp.float32),
    ],
)


def _tc2_body(agg_ref, cnt_ref, h1_ref, wr2_ref, bl2_ref, out_ref):
    inv = 1.0 / jnp.maximum(cnt_ref[0, :, 0:1] + cnt_ref[1, :, 0:1], 1.0)
    mean = jnp.concatenate([agg_ref[0], agg_ref[1]], axis=1) * inv
    out_ref[...] = (mean + bl2_ref[...]
                    + jnp.dot(h1_ref[...], wr2_ref[...],
                              preferred_element_type=jnp.float32))


_tc2 = pl.pallas_call(
    _tc2_body,
    grid=(_TCGRID,),
    in_specs=[
        pl.BlockSpec((2, _BT, HALF), lambda i: (0, i, 0)),
        pl.BlockSpec((2, _BT, 16), lambda i: (0, i, 0)),
        pl.BlockSpec((_BT, HID), lambda i: (i, 0)),
        pl.BlockSpec((HID, OUT), lambda i: (0, 0)),
        pl.BlockSpec((1, OUT), lambda i: (0, 0)),
    ],
    out_specs=pl.BlockSpec((_BT, OUT), lambda i: (i, 0)),
    out_shape=jax.ShapeDtypeStruct((ROWS, OUT), jnp.float32),
)


@jax.jit
def kernel(edge_index, user_emb, movie_emb, Wl1, bl1, Wr1, Wl2, bl2, Wr2):
    f32 = jnp.float32
    src = edge_index[0]
    dst = edge_index[1]

    npad = E_PAD - E
    ar = jnp.arange(npad, dtype=jnp.int32)
    src_p = jnp.concatenate([src, (ar * 997) % N_NODES])
    dst_p = jnp.concatenate([dst, N_NODES + (ar % (ROWS - N_NODES))])
    src2a = src_p.reshape(-1, CHUNK)
    dst2 = dst_p.reshape(-1, CHUNK)

    x = jnp.concatenate(
        [user_emb, movie_emb, jnp.zeros((ROWS - N_NODES, IN_CH), f32)], axis=0)
    x2 = jnp.stack([x[:, :HALF], x[:, HALF:]]).reshape(2 * ROWS, HALF)

    z32 = jnp.zeros((RPT, HALF), f32)
    z16 = jnp.zeros((RPT, 16), f32)
    o16 = jnp.ones((CHUNK, 16), f32)

    s_st = jnp.stack([src2a, src2a + ROWS])
    (cnt8,) = _sc_count(dst2, z16, o16)
    (agg1,) = _sc_pass(x2, s_st, dst2, z32)
    h1, y2 = _tc1(agg1, cnt8, x, Wl1.T, bl1.reshape(1, -1), Wr1.T, Wl2.T)
    (agg2,) = _sc_pass(y2.reshape(2 * ROWS, HALF), s_st, dst2, z32)
    out = _tc2(agg2, cnt8, h1, Wr2.T, bl2.reshape(1, -1))
    return out[:N_USERS], out[N_USERS:N_NODES]


# speedup vs baseline: 10.9930x; 1.0972x over previous
import functools

import jax
import jax.numpy as jnp
from jax import lax
from jax.experimental import pallas as pl
from jax.experimental.pallas import tpu as pltpu
from jax.experimental.pallas import tpu_sc as plsc

N_USERS = 25000
N_NODES = 50000
E = 800000
IN_CH = 64
HID = 128
OUT = 64

NT = 16
ROWS = 50048
RPT = ROWS // NT
CHUNK = 256
GC = 196
EPT = CHUNK * GC
E_PAD = EPT * NT
HALF = 32

_MESH = plsc.VectorSubcoreMesh(core_axis_name="c", subcore_axis_name="s")

NW = 2 * NT
GC2 = E_PAD // (CHUNK * NW)


_NB = 2


def _sc_pass_body(table, s_st, d2, z32, out2, acc, *scr):
    sis = (scr[0:_NB], scr[_NB:2 * _NB])
    dis = (scr[2 * _NB:3 * _NB], scr[3 * _NB:4 * _NB])
    rows = scr[4 * _NB:5 * _NB]
    gis = scr[5 * _NB:6 * _NB]
    gss = scr[6 * _NB:7 * _NB]
    gr = scr[7 * _NB]

    c = lax.axis_index("c")
    t = lax.axis_index("s")
    tslice = pl.ds(t * RPT, RPT)

    pltpu.sync_copy(z32, acc.at[tslice])
    plsc.subcore_barrier()

    NR = GC // _NB
    base = t * GC
    for b in range(_NB):
        pltpu.async_copy(s_st.at[c, base + b], sis[0][b], gis[b])
        pltpu.async_copy(d2.at[base + b], dis[0][b], gis[b])

    def round_(i, ii, ph, first, last):
        rb = base + _NB * i
        sc_, dc = sis[ph], dis[ph]
        sp, dp = sis[1 - ph], dis[1 - ph]
        gdescs = []
        for b in range(_NB):
            pltpu.make_async_copy(s_st.at[c, rb + b], sc_[b], gis[b]).wait()
            pltpu.make_async_copy(d2.at[rb + b], dc[b], gis[b]).wait()

            def drain_s():
                pltpu.make_async_copy(rows[b], acc.at[dp[b]], gss[b]).wait()
            if first is None:
                drain_s()
            else:
                pl.when(jnp.logical_not(first))(drain_s)

            def pref():
                rb2 = rb + _NB
                pltpu.async_copy(s_st.at[c, rb2 + b], sp[b], gis[b])
                pltpu.async_copy(d2.at[rb2 + b], dp[b], gis[b])
            if last is None:
                pref()
            else:
                pl.when(jnp.logical_not(last))(pref)

            gdescs.append(pltpu.async_copy(table.at[sc_[b]], rows[b], gr))
        for b in range(_NB):
            gdescs[b].wait()
            pltpu.async_copy(rows[b], acc.at[dc[b]], gss[b], add=True)

    @pl.loop(0, NR // 2)
    def _(ii):
        i = 2 * ii
        round_(i, ii, 0, first=(ii == 0), last=None)
        round_(i + 1, ii, 1, first=None, last=(ii == NR // 2 - 1))

    for b in range(_NB):
        pltpu.make_async_copy(rows[b], acc.at[dis[1][b]], gss[b]).wait()

    plsc.subcore_barrier()
    pltpu.sync_copy(acc.at[tslice], out2.at[c, tslice])


_sc_pass = pl.kernel(
    _sc_pass_body,
    out_type=[jax.ShapeDtypeStruct((2, ROWS, HALF), jnp.float32)],
    mesh=_MESH,
    scratch_types=(
        [pltpu.VMEM_SHARED((ROWS, HALF), jnp.float32)]
        + [pltpu.VMEM((CHUNK,), jnp.int32) for _ in range(4 * _NB)]
        + [pltpu.VMEM((CHUNK, HALF), jnp.float32) for _ in range(_NB)]
        + [pltpu.SemaphoreType.DMA for _ in range(2 * _NB + 1)]
    ),
    compiler_params=pltpu.CompilerParams(use_tc_tiling_on_sc=False),
    name="sage_seg_sum",
)


def _sc_count_body(d2, z16, o16, cntout, cnt, di0, ones_v):
    c = lax.axis_index("c")
    t = lax.axis_index("s")
    tslice = pl.ds(t * RPT, RPT)

    pltpu.sync_copy(z16, cnt.at[tslice])
    pltpu.sync_copy(o16, ones_v)
    plsc.subcore_barrier()

    @pl.loop(0, GC2)
    def _(chunk):
        rb = (c * NT + t) * GC2 + chunk
        pltpu.sync_copy(d2.at[rb], di0)
        pltpu.sync_copy(ones_v, cnt.at[di0], add=True)

    plsc.subcore_barrier()
    pltpu.sync_copy(cnt.at[tslice], cntout.at[c, tslice])


_sc_count = pl.kernel(
    _sc_count_body,
    out_type=[jax.ShapeDtypeStruct((2, ROWS, 16), jnp.float32)],
    mesh=_MESH,
    scratch_types=[
        pltpu.VMEM_SHARED((ROWS, 16), jnp.float32),
        pltpu.VMEM((CHUNK,), jnp.int32),
        pltpu.VMEM((CHUNK, 16), jnp.float32),
    ],
    compiler_params=pltpu.CompilerParams(use_tc_tiling_on_sc=False),
    name="sage_seg_count",
)


_BT = 3128
_TCGRID = ROWS // _BT


def _tc1_body(agg_ref, cnt_ref, x_ref, wl1_ref, bl1_ref, wr1_ref, wl2_ref,
              h1_ref, y2_ref):
    inv = 1.0 / jnp.maximum(cnt_ref[0, :, 0:1] + cnt_ref[1, :, 0:1], 1.0)
    mean = jnp.concatenate([agg_ref[0], agg_ref[1]], axis=1) * inv
    pre = (jnp.dot(mean, wl1_ref[...], preferred_element_type=jnp.float32)
           + bl1_ref[...]
           + jnp.dot(x_ref[...], wr1_ref[...], preferred_element_type=jnp.float32))
    h1 = jnp.maximum(pre, 0.0)
    h1_ref[...] = h1
    y = jnp.dot(h1, wl2_ref[...], preferred_element_type=jnp.float32)
    y2_ref[0] = y[:, :HALF]
    y2_ref[1] = y[:, HALF:]


_tc1 = pl.pallas_call(
    _tc1_body,
    grid=(_TCGRID,),
    in_specs=[
        pl.BlockSpec((2, _BT, HALF), lambda i: (0, i, 0)),
        pl.BlockSpec((2, _BT, 16), lambda i: (0, i, 0)),
        pl.BlockSpec((_BT, IN_CH), lambda i: (i, 0)),
        pl.BlockSpec((IN_CH, HID), lambda i: (0, 0)),
        pl.BlockSpec((1, HID), lambda i: (0, 0)),
        pl.BlockSpec((IN_CH, HID), lambda i: (0, 0)),
        pl.BlockSpec((HID, OUT), lambda i: (0, 0)),
    ],
    out_specs=[
        pl.BlockSpec((_BT, HID), lambda i: (i, 0)),
        pl.BlockSpec((2, _BT, HALF), lambda i: (0, i, 0)),
    ],
    out_shape=[
        jax.ShapeDtypeStruct((ROWS, HID), jnp.float32),
        jax.ShapeDtypeStruct((2, ROWS, HALF), jnp.float32),
    ],
)


def _tc2_body(agg_ref, cnt_ref, h1_ref, wr2_ref, bl2_ref, out_ref):
    inv = 1.0 / jnp.maximum(cnt_ref[0, :, 0:1] + cnt_ref[1, :, 0:1], 1.0)
    mean = jnp.concatenate([agg_ref[0], agg_ref[1]], axis=1) * inv
    out_ref[...] = (mean + bl2_ref[...]
                    + jnp.dot(h1_ref[...], wr2_ref[...],
                              preferred_element_type=jnp.float32))


_tc2 = pl.pallas_call(
    _tc2_body,
    grid=(_TCGRID,),
    in_specs=[
        pl.BlockSpec((2, _BT, HALF), lambda i: (0, i, 0)),
        pl.BlockSpec((2, _BT, 16), lambda i: (0, i, 0)),
        pl.BlockSpec((_BT, HID), lambda i: (i, 0)),
        pl.BlockSpec((HID, OUT), lambda i: (0, 0)),
        pl.BlockSpec((1, OUT), lambda i: (0, 0)),
    ],
    out_specs=pl.BlockSpec((_BT, OUT), lambda i: (i, 0)),
    out_shape=jax.ShapeDtypeStruct((ROWS, OUT), jnp.float32),
)


@jax.jit
def kernel(edge_index, user_emb, movie_emb, Wl1, bl1, Wr1, Wl2, bl2, Wr2):
    f32 = jnp.float32
    src = edge_index[0]
    dst = edge_index[1]

    npad = E_PAD - E
    ar = jnp.arange(npad, dtype=jnp.int32)
    src_p = jnp.concatenate([src, (ar * 997) % N_NODES])
    dst_p = jnp.concatenate([dst, N_NODES + (ar % (ROWS - N_NODES))])
    src2a = src_p.reshape(-1, CHUNK)
    dst2 = dst_p.reshape(-1, CHUNK)

    x = jnp.concatenate(
        [user_emb, movie_emb, jnp.zeros((ROWS - N_NODES, IN_CH), f32)], axis=0)
    x2 = jnp.stack([x[:, :HALF], x[:, HALF:]]).reshape(2 * ROWS, HALF)

    z32 = jnp.zeros((RPT, HALF), f32)
    z16 = jnp.zeros((RPT, 16), f32)
    o16 = jnp.ones((CHUNK, 16), f32)

    s_st = jnp.stack([src2a, src2a + ROWS])
    (cnt8,) = _sc_count(dst2, z16, o16)
    (agg1,) = _sc_pass(x2, s_st, dst2, z32)
    h1, y2 = _tc1(agg1, cnt8, x, Wl1.T, bl1.reshape(1, -1), Wr1.T, Wl2.T)
    (agg2,) = _sc_pass(y2.reshape(2 * ROWS, HALF), s_st, dst2, z32)
    out = _tc2(agg2, cnt8, h1, Wr2.T, bl2.reshape(1, -1))
    return out[:N_USERS], out[N_USERS:N_NODES]


# speedup vs baseline: 11.4953x; 1.0457x over previous
import functools

import jax
import jax.numpy as jnp
from jax import lax
from jax.experimental import pallas as pl
from jax.experimental.pallas import tpu as pltpu
from jax.experimental.pallas import tpu_sc as plsc

N_USERS = 25000
N_NODES = 50000
E = 800000
IN_CH = 64
HID = 128
OUT = 64

NT = 16
ROWS = 50048
RPT = ROWS // NT
CHUNK = 384
GC = 132
EPT = CHUNK * GC
E_PAD = EPT * NT
HALF = 32

_MESH = plsc.VectorSubcoreMesh(core_axis_name="c", subcore_axis_name="s")

NW = 2 * NT
GC2 = E_PAD // (CHUNK * NW)


_NB = 2


def _sc_pass_body(table, s_st, d2, z32, out2, acc, *scr):
    sis = (scr[0:_NB], scr[_NB:2 * _NB])
    dis = (scr[2 * _NB:3 * _NB], scr[3 * _NB:4 * _NB])
    rows = scr[4 * _NB:5 * _NB]
    gis = scr[5 * _NB:6 * _NB]
    gss = scr[6 * _NB:7 * _NB]
    gr = scr[7 * _NB]

    c = lax.axis_index("c")
    t = lax.axis_index("s")
    tslice = pl.ds(t * RPT, RPT)

    pltpu.sync_copy(z32, acc.at[tslice])
    plsc.subcore_barrier()

    NR = GC // _NB
    base = t * GC
    for b in range(_NB):
        pltpu.async_copy(s_st.at[c, base + b], sis[0][b], gis[b])
        pltpu.async_copy(d2.at[base + b], dis[0][b], gis[b])

    def round_(i, ii, ph, first, last):
        rb = base + _NB * i
        sc_, dc = sis[ph], dis[ph]
        sp, dp = sis[1 - ph], dis[1 - ph]
        gdescs = []
        for b in range(_NB):
            pltpu.make_async_copy(s_st.at[c, rb + b], sc_[b], gis[b]).wait()
            pltpu.make_async_copy(d2.at[rb + b], dc[b], gis[b]).wait()

            def drain_s():
                pltpu.make_async_copy(rows[b], acc.at[dp[b]], gss[b]).wait()
            if first is None:
                drain_s()
            else:
                pl.when(jnp.logical_not(first))(drain_s)

            def pref():
                rb2 = rb + _NB
                pltpu.async_copy(s_st.at[c, rb2 + b], sp[b], gis[b])
                pltpu.async_copy(d2.at[rb2 + b], dp[b], gis[b])
            if last is None:
                pref()
            else:
                pl.when(jnp.logical_not(last))(pref)

            gdescs.append(pltpu.async_copy(table.at[sc_[b]], rows[b], gr))
        for b in range(_NB):
            gdescs[b].wait()
            pltpu.async_copy(rows[b], acc.at[dc[b]], gss[b], add=True)

    @pl.loop(0, NR // 2)
    def _(ii):
        i = 2 * ii
        round_(i, ii, 0, first=(ii == 0), last=None)
        round_(i + 1, ii, 1, first=None, last=(ii == NR // 2 - 1))

    for b in range(_NB):
        pltpu.make_async_copy(rows[b], acc.at[dis[1][b]], gss[b]).wait()

    plsc.subcore_barrier()
    pltpu.sync_copy(acc.at[tslice], out2.at[c, tslice])


_sc_pass = pl.kernel(
    _sc_pass_body,
    out_type=[jax.ShapeDtypeStruct((2, ROWS, HALF), jnp.float32)],
    mesh=_MESH,
    scratch_types=(
        [pltpu.VMEM_SHARED((ROWS, HALF), jnp.float32)]
        + [pltpu.VMEM((CHUNK,), jnp.int32) for _ in range(4 * _NB)]
        + [pltpu.VMEM((CHUNK, HALF), jnp.float32) for _ in range(_NB)]
        + [pltpu.SemaphoreType.DMA for _ in range(2 * _NB + 1)]
    ),
    compiler_params=pltpu.CompilerParams(use_tc_tiling_on_sc=False),
    name="sage_seg_sum",
)


def _sc_count_body(d2, z16, o16, cntout, cnt, di0, ones_v):
    c = lax.axis_index("c")
    t = lax.axis_index("s")
    tslice = pl.ds(t * RPT, RPT)

    pltpu.sync_copy(z16, cnt.at[tslice])
    pltpu.sync_copy(o16, ones_v)
    plsc.subcore_barrier()

    @pl.loop(0, GC2)
    def _(chunk):
        rb = (c * NT + t) * GC2 + chunk
        pltpu.sync_copy(d2.at[rb], di0)
        pltpu.sync_copy(ones_v, cnt.at[di0], add=True)

    plsc.subcore_barrier()
    pltpu.sync_copy(cnt.at[tslice], cntout.at[c, tslice])


_sc_count = pl.kernel(
    _sc_count_body,
    out_type=[jax.ShapeDtypeStruct((2, ROWS, 16), jnp.float32)],
    mesh=_MESH,
    scratch_types=[
        pltpu.VMEM_SHARED((ROWS, 16), jnp.float32),
        pltpu.VMEM((CHUNK,), jnp.int32),
        pltpu.VMEM((CHUNK, 16), jnp.float32),
    ],
    compiler_params=pltpu.CompilerParams(use_tc_tiling_on_sc=False),
    name="sage_seg_count",
)


_BT = 3128
_TCGRID = ROWS // _BT


def _tc1_body(agg_ref, cnt_ref, x_ref, wl1_ref, bl1_ref, wr1_ref, wl2_ref,
              h1_ref, y2_ref):
    inv = 1.0 / jnp.maximum(cnt_ref[0, :, 0:1] + cnt_ref[1, :, 0:1], 1.0)
    mean = jnp.concatenate([agg_ref[0], agg_ref[1]], axis=1) * inv
    pre = (jnp.dot(mean, wl1_ref[...], preferred_element_type=jnp.float32)
           + bl1_ref[...]
           + jnp.dot(x_ref[...], wr1_ref[...], preferred_element_type=jnp.float32))
    h1 = jnp.maximum(pre, 0.0)
    h1_ref[...] = h1
    y = jnp.dot(h1, wl2_ref[...], preferred_element_type=jnp.float32)
    y2_ref[0] = y[:, :HALF]
    y2_ref[1] = y[:, HALF:]


_tc1 = pl.pallas_call(
    _tc1_body,
    grid=(_TCGRID,),
    in_specs=[
        pl.BlockSpec((2, _BT, HALF), lambda i: (0, i, 0)),
        pl.BlockSpec((2, _BT, 16), lambda i: (0, i, 0)),
        pl.BlockSpec((_BT, IN_CH), lambda i: (i, 0)),
        pl.BlockSpec((IN_CH, HID), lambda i: (0, 0)),
        pl.BlockSpec((1, HID), lambda i: (0, 0)),
        pl.BlockSpec((IN_CH, HID), lambda i: (0, 0)),
        pl.BlockSpec((HID, OUT), lambda i: (0, 0)),
    ],
    out_specs=[
        pl.BlockSpec((_BT, HID), lambda i: (i, 0)),
        pl.BlockSpec((2, _BT, HALF), lambda i: (0, i, 0)),
    ],
    out_shape=[
        jax.ShapeDtypeStruct((ROWS, HID), jnp.float32),
        jax.ShapeDtypeStruct((2, ROWS, HALF), jnp.float32),
    ],
)


def _tc2_body(agg_ref, cnt_ref, h1_ref, wr2_ref, bl2_ref, out_ref):
    inv = 1.0 / jnp.maximum(cnt_ref[0, :, 0:1] + cnt_ref[1, :, 0:1], 1.0)
    mean = jnp.concatenate([agg_ref[0], agg_ref[1]], axis=1) * inv
    out_ref[...] = (mean + bl2_ref[...]
                    + jnp.dot(h1_ref[...], wr2_ref[...],
                              preferred_element_type=jnp.float32))


_tc2 = pl.pallas_call(
    _tc2_body,
    grid=(_TCGRID,),
    in_specs=[
        pl.BlockSpec((2, _BT, HALF), lambda i: (0, i, 0)),
        pl.BlockSpec((2, _BT, 16), lambda i: (0, i, 0)),
        pl.BlockSpec((_BT, HID), lambda i: (i, 0)),
        pl.BlockSpec((HID, OUT), lambda i: (0, 0)),
        pl.BlockSpec((1, OUT), lambda i: (0, 0)),
    ],
    out_specs=pl.BlockSpec((_BT, OUT), lambda i: (i, 0)),
    out_shape=jax.ShapeDtypeStruct((ROWS, OUT), jnp.float32),
)


@jax.jit
def kernel(edge_index, user_emb, movie_emb, Wl1, bl1, Wr1, Wl2, bl2, Wr2):
    f32 = jnp.float32
    src = edge_index[0]
    dst = edge_index[1]

    npad = E_PAD - E
    ar = jnp.arange(npad, dtype=jnp.int32)
    src_p = jnp.concatenate([src, (ar * 997) % N_NODES])
    dst_p = jnp.concatenate([dst, N_NODES + (ar % (ROWS - N_NODES))])
    src2a = src_p.reshape(-1, CHUNK)
    dst2 = dst_p.reshape(-1, CHUNK)

    x = jnp.concatenate(
        [user_emb, movie_emb, jnp.zeros((ROWS - N_NODES, IN_CH), f32)], axis=0)
    x2 = jnp.stack([x[:, :HALF], x[:, HALF:]]).reshape(2 * ROWS, HALF)

    z32 = jnp.zeros((RPT, HALF), f32)
    z16 = jnp.zeros((RPT, 16), f32)
    o16 = jnp.ones((CHUNK, 16), f32)

    s_st = jnp.stack([src2a, src2a + ROWS])
    (cnt8,) = _sc_count(dst2, z16, o16)
    (agg1,) = _sc_pass(x2, s_st, dst2, z32)
    h1, y2 = _tc1(agg1, cnt8, x, Wl1.T, bl1.reshape(1, -1), Wr1.T, Wl2.T)
    (agg2,) = _sc_pass(y2.reshape(2 * ROWS, HALF), s_st, dst2, z32)
    out = _tc2(agg2, cnt8, h1, Wr2.T, bl2.reshape(1, -1))
    return out[:N_USERS], out[N_USERS:N_NODES]
